# baseline (speedup 1.0000x reference)
import jax
import jax.numpy as jnp
from jax import lax
from jax.experimental import pallas as pl
from jax.experimental.pallas import tpu as pltpu
from jax.experimental.pallas import tpu_sc as plsc

VOCAB = 40000
HIDDEN = 312
DPAD = 384
MAX_POS = 2048
B, L = 32, 2048
TOKENS = B * L
EPS = 1e-12

NC, NS = 2, 16
NW = NC * NS
TPW = TOKENS // NW
C = 64
NCHUNK = TPW // C
NPAIR = NCHUNK // 2
NVREG = DPAD // 16
NCOMBO = 64
CPAD = 320
NCV = CPAD // 16


def _rsqrt(x):
    i = lax.bitcast_convert_type(x, jnp.int32)
    i = jnp.int32(0x5F3759DF) - lax.shift_right_logical(i, 1)
    y = lax.bitcast_convert_type(i, jnp.float32)
    for _ in range(3):
        y = y * (1.5 - 0.5 * x * y * y)
    return y


def _allreduce_sum(v, perms):
    for idx in perms:
        v = v + v.at[idx].get(mode="promise_in_bounds")
    return v


def _body(ids, pos, tok, task, w_word, w_pos, w_tok, w_task, g, b, out,
          iw0, ip0, it0, ik0, iw1, ip1, it1, ik1,
          bw0, bp0, bw1, bp1,
          tk_v, combo, g2, b2, cb_v,
          gsem0, gsem1, wsem0, wsem1):
    wid = lax.axis_index("s") * NC + lax.axis_index("c")
    base = wid * TPW
    pltpu.sync_copy(g, g2)
    pltpu.sync_copy(b, b2)
    pltpu.sync_copy(w_tok, tk_v)
    pltpu.sync_copy(w_task, bp0.at[pl.ds(0, 16)])
    lane = lax.iota(jnp.int32, 16)
    perms = [lane ^ m for m in (1, 2, 4, 8)]

    okv = jnp.ones((16,), jnp.bool_)
    for k in range(NCV - 1):
        sl = pl.ds(16 * k, 16)
        okv = okv & (g2[sl] == 1.0) & (b2[sl] == 0.0)
    tl = pl.ds(HIDDEN - 16, 16)
    okv = okv & ((g2[tl] == 1.0) & (b2[tl] == 0.0) | (lane < 8))
    oki = jnp.where(okv, jnp.int32(1), jnp.int32(0))
    for idx in perms:
        oki = oki & oki.at[idx].get(mode="promise_in_bounds")
    trivial_affine = oki[0] == 1

    def combo_body(j, carry):
        t = lax.shift_right_logical(j, 4)
        k2 = lax.bitwise_and(j, 15)
        for k in range(NCV):
            sl = pl.ds(16 * k, 16)
            combo[pl.ds(j * CPAD + 16 * k, 16)] = tk_v[t, sl] + bp0[k2, sl]
        return carry

    lax.fori_loop(0, NCOMBO, combo_body, 0, unroll=False)

    iws = ((iw0, ip0, it0, ik0), (iw1, ip1, it1, ik1))
    bufs = ((bw0, bp0), (bw1, bp1))
    gsems = (gsem0, gsem1)
    wsems = (wsem0, wsem1)
    tables = (w_word, w_pos)
    streams = (ids, pos, tok, task)

    def issue_gathers(c, s):
        row0 = base + c * C
        for st, ib in zip(streams, iws[s]):
            pltpu.sync_copy(st.at[pl.ds(row0, C)], ib)
        for tb, ib, bf in zip(tables, iws[s], bufs[s]):
            pltpu.async_copy(tb.at[ib], bf, gsems[s])

    def wait_gathers(s):
        for tb, ib, bf in zip(tables, iws[s], bufs[s]):
            pltpu.make_async_copy(tb.at[ib], bf, gsems[s]).wait()

    def issue_write(c, s):
        row0 = base + c * C
        pltpu.async_copy(bufs[s][0], out.at[pl.ds(row0, C)], wsems[s])

    def wait_write(s):
        pltpu.make_async_copy(bufs[s][0], out.at[pl.ds(base, C)],
                              wsems[s]).wait()

    def compute_chunk(s):
        bw, bp = bufs[s]
        it_r, ik_r = iws[s][2], iws[s][3]
        for q in range(C // 16):
            sl = pl.ds(16 * q, 16)
            cb_v[sl] = (it_r[sl] * 16 + ik_r[sl]) * CPAD

        @plsc.parallel_loop(0, C, 1, unroll=2)
        def row_body(r):
            rsp = lax.broadcast(r, (16,))
            cbase = plsc.load_gather(cb_v, [rsp]) + lane
            vs = []
            sv = [jnp.zeros((16,), jnp.float32) for _ in range(2)]
            sq = [jnp.zeros((16,), jnp.float32) for _ in range(2)]
            for k in range(NCV):
                sl = pl.ds(16 * k, 16)
                v = bw[r, sl] + bp[r, sl]
                v = v + plsc.load_gather(combo, [cbase + 16 * k])
                vs.append(v)
                sv[k % 2] = sv[k % 2] + v
                sq[k % 2] = sq[k % 2] + v * v
            ssum = _allreduce_sum(sv[0] + sv[1], perms)
            qsum = _allreduce_sum(sq[0] + sq[1], perms)
            mean = ssum * (1.0 / HIDDEN)
            var = jnp.maximum(qsum * (1.0 / HIDDEN) - mean * mean, 0.0)
            rs = _rsqrt(var + EPS)
            @pl.when(trivial_affine)
            def _():
                for k in range(NCV):
                    sl = pl.ds(16 * k, 16)
                    bw[r, sl] = (vs[k] - mean) * rs

            @pl.when(jnp.logical_not(trivial_affine))
            def _():
                for k in range(NCV):
                    sl = pl.ds(16 * k, 16)
                    bw[r, sl] = (vs[k] - mean) * rs * g2[sl] + b2[sl]

    issue_gathers(0, 0)

    def pair_body(i, carry):
        e = 2 * i

        @pl.when(i > 0)
        def _():
            wait_write(1)

        issue_gathers(e + 1, 1)
        wait_gathers(0)
        compute_chunk(0)
        issue_write(e, 0)
        wait_write(0)

        @pl.when(i < NPAIR - 1)
        def _():
            issue_gathers(e + 2, 0)

        wait_gathers(1)
        compute_chunk(1)
        issue_write(e + 1, 1)
        return carry

    lax.fori_loop(0, NPAIR, pair_body, 0, unroll=False)
    wait_write(1)


_scratch = (
    [pltpu.VMEM((C,), jnp.int32)] * 8
    + [pltpu.VMEM((C, DPAD), jnp.float32)] * 4
    + [pltpu.VMEM((4, DPAD), jnp.float32),
       pltpu.VMEM((NCOMBO * CPAD,), jnp.float32),
       pltpu.VMEM((DPAD,), jnp.float32),
       pltpu.VMEM((DPAD,), jnp.float32),
       pltpu.VMEM((C,), jnp.int32)]
    + [pltpu.SemaphoreType.DMA] * 4
)

_emb_kernel = pl.kernel(
    _body,
    out_type=jax.ShapeDtypeStruct((TOKENS, DPAD), jnp.float32),
    mesh=plsc.VectorSubcoreMesh(core_axis_name="c", subcore_axis_name="s"),
    compiler_params=pltpu.CompilerParams(use_tc_tiling_on_sc=True,
                                         needs_layout_passes=False),
    scratch_types=list(_scratch),
)


def _pad(w):
    return jnp.pad(w, ((0, 0), (0, DPAD - HIDDEN)))


def kernel(input_ids, position_ids, token_type_ids, task_type_ids,
           W_word, W_pos, W_tok, W_task, gamma, beta):
    ids = input_ids.reshape(TOKENS).astype(jnp.int32)
    pos = position_ids.reshape(TOKENS).astype(jnp.int32)
    tok = token_type_ids.reshape(TOKENS).astype(jnp.int32)
    task = task_type_ids.reshape(TOKENS).astype(jnp.int32)
    gp = jnp.pad(gamma, (0, DPAD - HIDDEN))
    bp_ = jnp.pad(beta, (0, DPAD - HIDDEN))
    out = _emb_kernel(ids, pos, tok, task, _pad(W_word), _pad(W_pos),
                      _pad(W_tok), _pad(W_task), gp, bp_)
    return out[:, :HIDDEN].reshape(B, L, HIDDEN)
